# Initial kernel scaffold; baseline (speedup 1.0000x reference)
import jax
import jax.numpy as jnp
from jax.experimental import pallas as pl


def kernel(tokens, table, ctx):
    raise NotImplementedError("write your pallas kernel here")


# speedup vs baseline: 1.0279x; 1.0279x over previous
import functools

import jax
import jax.numpy as jnp
from jax import lax
from jax.experimental import pallas as pl
from jax.experimental.pallas import tpu as pltpu
from jax.experimental.pallas import tpu_sc as plsc

VOCAB = 49408
K = 1000
N_TOK = 77
N_CTX = 16
DIM = 768

NW = 32
KPC = 32
K_PAD = NW * KPC
NG = N_TOK - N_CTX
IDX_W = 64


def _sc_body(idxr, table, ctx, out,
             idx_v, buf0, buf1, ctx_v,
             gsem0, gsem1, wsem0, wsem1, csem):
    wid = lax.axis_index("s") * 2 + lax.axis_index("c")
    base = wid * KPC
    pltpu.sync_copy(idxr.at[pl.ds(base, KPC)], idx_v)
    pltpu.sync_copy(ctx, ctx_v)

    def writeback(i, buf, wsem, issue):
        k = base + i
        ko = k * N_TOK
        c1 = pltpu.make_async_copy(buf.at[pl.ds(0, 1)], out.at[pl.ds(ko, 1)], wsem)
        c2 = pltpu.make_async_copy(ctx_v, out.at[pl.ds(ko + 1, N_CTX)], csem)
        c3 = pltpu.make_async_copy(buf.at[pl.ds(1, NG - 1)],
                                   out.at[pl.ds(ko + 1 + N_CTX, NG - 1)], wsem)
        for c in (c1, c2, c3):
            if issue:
                c.start()
            else:
                c.wait()

    def half(i, buf, gsem, wsem):
        k = base + i

        @pl.when(jnp.logical_and(i >= 2, (k - 2) < K))
        def _drain():
            writeback(i - 2, buf, wsem, issue=False)

        pltpu.async_copy(table.at[idx_v.at[i]], buf, gsem).wait()

        @pl.when(k < K)
        def _issue():
            writeback(i, buf, wsem, issue=True)

    def loop_body(j, carry):
        half(2 * j, buf0, gsem0, wsem0)
        half(2 * j + 1, buf1, gsem1, wsem1)
        return carry

    lax.fori_loop(0, KPC // 2, loop_body, 0)

    for i, buf, wsem in ((KPC - 2, buf0, wsem0), (KPC - 1, buf1, wsem1)):
        @pl.when((base + i) < K)
        def _final_drain(i=i, buf=buf, wsem=wsem):
            writeback(i, buf, wsem, issue=False)


def kernel(tokens, table, ctx):
    tokens = tokens.astype(jnp.int32)
    idxr = jnp.zeros((K_PAD, IDX_W), jnp.int32)
    idxr = idxr.at[:K, 0].set(tokens[:, 0])
    idxr = idxr.at[:K, 1:NG].set(tokens[:, 1 + N_CTX:])

    mesh = plsc.VectorSubcoreMesh(core_axis_name="c", subcore_axis_name="s")
    run = pl.kernel(
        _sc_body,
        out_type=jax.ShapeDtypeStruct((K * N_TOK, DIM), jnp.float32),
        mesh=mesh,
        compiler_params=pltpu.CompilerParams(use_tc_tiling_on_sc=False),
        scratch_types=[
            pltpu.VMEM((KPC, IDX_W), jnp.int32),
            pltpu.VMEM((IDX_W, DIM), jnp.float32),
            pltpu.VMEM((IDX_W, DIM), jnp.float32),
            pltpu.VMEM((N_CTX, DIM), jnp.float32),
            pltpu.SemaphoreType.DMA,
            pltpu.SemaphoreType.DMA,
            pltpu.SemaphoreType.DMA,
            pltpu.SemaphoreType.DMA,
            pltpu.SemaphoreType.DMA,
        ],
    )
    out = run(idxr, table, ctx)
    return out.reshape(K, N_TOK, DIM)


# speedup vs baseline: 1.0305x; 1.0025x over previous
import functools

import jax
import jax.numpy as jnp
from jax import lax
from jax.experimental import pallas as pl
from jax.experimental.pallas import tpu as pltpu
from jax.experimental.pallas import tpu_sc as plsc

VOCAB = 49408
K = 1000
N_TOK = 77
N_CTX = 16
DIM = 768

NW = 32
KPC = 32
K_PAD = NW * KPC
NG = N_TOK - N_CTX
IDX_W = 64


def _sc_body(idxr, table, ctx, out,
             idx_v, buf0, buf1, ctx_v,
             gsem0, gsem1, wsem0, wsem1, csem):
    wid = lax.axis_index("s") * 2 + lax.axis_index("c")
    base = wid * KPC
    pltpu.sync_copy(idxr.at[pl.ds(base, KPC)], idx_v)
    pltpu.sync_copy(ctx, ctx_v)

    def writeback(i, buf, wsem, issue):
        k = base + i
        c1 = pltpu.make_async_copy(buf.at[pl.ds(0, 1)], out.at[k, pl.ds(0, 1)], wsem)
        c2 = pltpu.make_async_copy(ctx_v, out.at[k, pl.ds(1, N_CTX)], csem)
        c3 = pltpu.make_async_copy(buf.at[pl.ds(1, NG - 1)],
                                   out.at[k, pl.ds(1 + N_CTX, NG - 1)], wsem)
        for c in (c1, c2, c3):
            if issue:
                c.start()
            else:
                c.wait()

    def half(i, buf, gsem, wsem):
        k = base + i

        @pl.when(jnp.logical_and(i >= 2, (k - 2) < K))
        def _drain():
            writeback(i - 2, buf, wsem, issue=False)

        pltpu.async_copy(table.at[idx_v.at[i]], buf, gsem).wait()

        @pl.when(k < K)
        def _issue():
            writeback(i, buf, wsem, issue=True)

    def loop_body(j, carry):
        half(2 * j, buf0, gsem0, wsem0)
        half(2 * j + 1, buf1, gsem1, wsem1)
        return carry

    lax.fori_loop(0, KPC // 2, loop_body, 0)

    for i, buf, wsem in ((KPC - 2, buf0, wsem0), (KPC - 1, buf1, wsem1)):
        @pl.when((base + i) < K)
        def _final_drain(i=i, buf=buf, wsem=wsem):
            writeback(i, buf, wsem, issue=False)


def kernel(tokens, table, ctx):
    tokens = tokens.astype(jnp.int32)
    idxr = jnp.zeros((K_PAD, IDX_W), jnp.int32)
    idxr = idxr.at[:K, 0].set(tokens[:, 0])
    idxr = idxr.at[:K, 1:NG].set(tokens[:, 1 + N_CTX:])

    mesh = plsc.VectorSubcoreMesh(core_axis_name="c", subcore_axis_name="s")
    run = pl.kernel(
        _sc_body,
        out_type=jax.ShapeDtypeStruct((K, N_TOK, DIM), jnp.float32),
        mesh=mesh,
        compiler_params=pltpu.CompilerParams(use_tc_tiling_on_sc=False),
        scratch_types=[
            pltpu.VMEM((KPC, IDX_W), jnp.int32),
            pltpu.VMEM((IDX_W, DIM), jnp.float32),
            pltpu.VMEM((IDX_W, DIM), jnp.float32),
            pltpu.VMEM((N_CTX, DIM), jnp.float32),
            pltpu.SemaphoreType.DMA,
            pltpu.SemaphoreType.DMA,
            pltpu.SemaphoreType.DMA,
            pltpu.SemaphoreType.DMA,
            pltpu.SemaphoreType.DMA,
        ],
    )
    return run(idxr, table, ctx)


# speedup vs baseline: 1.7907x; 1.7377x over previous
import functools

import jax
import jax.numpy as jnp
from jax import lax
from jax.experimental import pallas as pl
from jax.experimental.pallas import tpu as pltpu
from jax.experimental.pallas import tpu_sc as plsc

VOCAB = 49408
K = 1000
N_TOK = 77
N_CTX = 16
DIM = 768

NW = 32
CH = 50
CW = 56
NCH = (K * (N_TOK - N_CTX)) // CH
NCH_PAD = 1232
NC_MAX = 39
CTX_ROWS = (K * N_CTX) // NW
CTX_REP = 50


def _sc_body(idxg, table, ctx, out,
             idx_v, buf0, buf1, cbuf,
             gsem0, gsem1, wsem0, wsem1, csem):
    wid = lax.axis_index("s") * 2 + lax.axis_index("c")
    base_c = wid * 38 + jnp.minimum(wid, 4)
    nc = 38 + jnp.where(wid < 4, 1, 0)
    pltpu.sync_copy(idxg.at[pl.ds(base_c, NC_MAX)], idx_v)

    pltpu.sync_copy(ctx.at[wid // 2], cbuf)
    r0 = K + CTX_ROWS * wid
    for m in range(CTX_ROWS // CTX_REP):
        pltpu.async_copy(cbuf, out.at[pl.ds(r0 + CTX_REP * m, CTX_REP)], csem)

    def dst_row(c):
        return CH * c + jnp.where(c >= K // CH, (N_CTX - 1) * K, 0)

    def chunk(i, buf, gsem, wsem):
        c = base_c + i

        @pl.when(i < nc)
        def _():
            @pl.when(i >= 2)
            def _drain():
                pltpu.make_async_copy(buf.at[pl.ds(0, CH)],
                                      out.at[pl.ds(dst_row(c - 2), CH)],
                                      wsem).wait()
            pltpu.async_copy(table.at[idx_v.at[i]], buf, gsem).wait()
            pltpu.async_copy(buf.at[pl.ds(0, CH)],
                             out.at[pl.ds(dst_row(c), CH)], wsem)

    def loop_body(j, carry):
        chunk(2 * j, buf0, gsem0, wsem0)
        chunk(2 * j + 1, buf1, gsem1, wsem1)
        return carry

    lax.fori_loop(0, (NC_MAX + 1) // 2, loop_body, 0)

    for buf, wsem in ((buf0, wsem0), (buf1, wsem1)):
        pltpu.make_async_copy(buf.at[pl.ds(0, CH)],
                              out.at[pl.ds(0, CH)], wsem).wait()
    for m in range(CTX_ROWS // CTX_REP):
        pltpu.make_async_copy(cbuf, out.at[pl.ds(0, CTX_REP)], csem).wait()


def kernel(tokens, table, ctx):
    tokens = tokens.astype(jnp.int32)
    idx_flat = jnp.concatenate(
        [tokens[:, 0], tokens[:, 1 + N_CTX:].T.reshape(-1)])
    idxg = jnp.zeros((NCH_PAD, CW), jnp.int32)
    idxg = idxg.at[:NCH, :CH].set(idx_flat.reshape(NCH, CH))
    ctx_rep = jnp.broadcast_to(ctx[:, None, :], (N_CTX, CTX_REP, DIM))

    mesh = plsc.VectorSubcoreMesh(core_axis_name="c", subcore_axis_name="s")
    run = pl.kernel(
        _sc_body,
        out_type=jax.ShapeDtypeStruct((N_TOK * K, DIM), jnp.float32),
        mesh=mesh,
        compiler_params=pltpu.CompilerParams(use_tc_tiling_on_sc=False),
        scratch_types=[
            pltpu.VMEM((NC_MAX, CW), jnp.int32),
            pltpu.VMEM((CW, DIM), jnp.float32),
            pltpu.VMEM((CW, DIM), jnp.float32),
            pltpu.VMEM((CTX_REP, DIM), jnp.float32),
            pltpu.SemaphoreType.DMA,
            pltpu.SemaphoreType.DMA,
            pltpu.SemaphoreType.DMA,
            pltpu.SemaphoreType.DMA,
            pltpu.SemaphoreType.DMA,
        ],
    )
    out = run(idxg, table, ctx_rep)
    return out.reshape(N_TOK, K, DIM).transpose(1, 0, 2)


# speedup vs baseline: 1.7955x; 1.0027x over previous
import functools

import jax
import jax.numpy as jnp
from jax import lax
from jax.experimental import pallas as pl
from jax.experimental.pallas import tpu as pltpu
from jax.experimental.pallas import tpu_sc as plsc

VOCAB = 49408
K = 1000
N_TOK = 77
N_CTX = 16
DIM = 768

NW = 32
CH = 50
CW = 56
NCH = (K * (N_TOK - N_CTX)) // CH
NCH_PAD = 1232
NC_MAX = 39
CTX_ROWS = (K * N_CTX) // NW
CTX_REP = 50


def _sc_body(idxg, table, ctx, out,
             idx_v, buf0, buf1, cbuf,
             gsem0, gsem1, wsem0, wsem1, csem):
    wid = lax.axis_index("s") * 2 + lax.axis_index("c")
    base_c = wid * 38 + jnp.minimum(wid, 4)
    nc = 38 + jnp.where(wid < 4, 1, 0)
    pltpu.sync_copy(idxg.at[pl.ds(base_c, NC_MAX)], idx_v)

    pltpu.sync_copy(ctx.at[wid // 2], cbuf)
    r0 = K + CTX_ROWS * wid
    for m in range(CTX_ROWS // CTX_REP):
        pltpu.async_copy(cbuf, out.at[pl.ds(r0 + CTX_REP * m, CTX_REP)], csem)

    def dst_row(c):
        return CH * c + jnp.where(c >= K // CH, N_CTX * K, 0)

    def chunk(i, buf, gsem, wsem):
        c = base_c + i

        @pl.when(i < nc)
        def _():
            @pl.when(i >= 2)
            def _drain():
                pltpu.make_async_copy(buf.at[pl.ds(0, CH)],
                                      out.at[pl.ds(dst_row(c - 2), CH)],
                                      wsem).wait()
            pltpu.async_copy(table.at[idx_v.at[i]], buf, gsem).wait()
            pltpu.async_copy(buf.at[pl.ds(0, CH)],
                             out.at[pl.ds(dst_row(c), CH)], wsem)

    def loop_body(j, carry):
        chunk(2 * j, buf0, gsem0, wsem0)
        chunk(2 * j + 1, buf1, gsem1, wsem1)
        return carry

    lax.fori_loop(0, (NC_MAX + 1) // 2, loop_body, 0)

    for buf, wsem in ((buf0, wsem0), (buf1, wsem1)):
        pltpu.make_async_copy(buf.at[pl.ds(0, CH)],
                              out.at[pl.ds(0, CH)], wsem).wait()
    for m in range(CTX_ROWS // CTX_REP):
        pltpu.make_async_copy(cbuf, out.at[pl.ds(0, CTX_REP)], csem).wait()


def kernel(tokens, table, ctx):
    tokens = tokens.astype(jnp.int32)
    idx_flat = jnp.concatenate(
        [tokens[:, 0], tokens[:, 1 + N_CTX:].T.reshape(-1)])
    idxg = jnp.zeros((NCH_PAD, CW), jnp.int32)
    idxg = idxg.at[:NCH, :CH].set(idx_flat.reshape(NCH, CH))
    ctx_rep = jnp.broadcast_to(ctx[:, None, :], (N_CTX, CTX_REP, DIM))

    mesh = plsc.VectorSubcoreMesh(core_axis_name="c", subcore_axis_name="s")
    run = pl.kernel(
        _sc_body,
        out_type=jax.ShapeDtypeStruct((N_TOK * K, DIM), jnp.float32),
        mesh=mesh,
        compiler_params=pltpu.CompilerParams(use_tc_tiling_on_sc=False),
        scratch_types=[
            pltpu.VMEM((NC_MAX, CW), jnp.int32),
            pltpu.VMEM((CW, DIM), jnp.float32),
            pltpu.VMEM((CW, DIM), jnp.float32),
            pltpu.VMEM((CTX_REP, DIM), jnp.float32),
            pltpu.SemaphoreType.DMA,
            pltpu.SemaphoreType.DMA,
            pltpu.SemaphoreType.DMA,
            pltpu.SemaphoreType.DMA,
            pltpu.SemaphoreType.DMA,
        ],
    )
    out = run(idxg, table, ctx_rep)
    return out.reshape(N_TOK, K, DIM).transpose(1, 0, 2)


# speedup vs baseline: 9.1796x; 5.1125x over previous
import functools

import jax
import jax.numpy as jnp
import numpy as np
from jax import lax
from jax.experimental import pallas as pl
from jax.experimental.pallas import tpu as pltpu
from jax.experimental.pallas import tpu_sc as plsc

VOCAB = 49408
K = 1000
N_TOK = 77
N_CTX = 16
DIM = 768

NW = 32
CH = 16
NCH = 3813
SCHUNK = 62
NC_MAX = 120
NBUF = 6
CTX_BLK = 40
CTX_NWR = K * N_CTX // (16 * CTX_BLK)


def _sc_body(idxw, table, ctxr, out, idx_v, cbuf, *scr):
    bufs = scr[:NBUF]
    gsems = scr[NBUF:2 * NBUF]
    wsems = scr[2 * NBUF:3 * NBUF]
    csem = scr[3 * NBUF]
    wid = lax.axis_index("s") * 2 + lax.axis_index("c")
    base = wid * 119 + jnp.minimum(wid, 5)
    nc = 119 + jnp.where(wid < 5, 1, 0)
    pltpu.sync_copy(idxw.at[wid], idx_v)

    @pl.when(wid < N_CTX)
    def _ctx():
        pltpu.sync_copy(ctxr.at[wid], cbuf)
        for m in range(CTX_NWR):
            r = pl.multiple_of(K * (wid + 1) + CTX_BLK * m, 8)
            pltpu.async_copy(cbuf, out.at[pl.ds(r, CTX_BLK)], csem)

    def writeback(c, buf, wsem, wait):
        r16 = pl.multiple_of(
            jnp.where(c < SCHUNK, CH * c, CH * c + N_CTX * K - 8), 8)
        d16 = pltpu.make_async_copy(buf, out.at[pl.ds(r16, CH)], wsem)
        d8 = pltpu.make_async_copy(buf.at[pl.ds(0, 8)],
                                   out.at[pl.ds(SCHUNK * CH, 8)], wsem)

        @pl.when(c == SCHUNK)
        def _():
            d8.wait() if wait else d8.start()

        @pl.when(c != SCHUNK)
        def _():
            d16.wait() if wait else d16.start()

    def step(i, h):
        @pl.when(jnp.logical_and(i >= 6, i - 6 < nc))
        def _drain():
            writeback(base + i - 6, bufs[h], wsems[h], wait=True)

        h3 = (h + 3) % NBUF

        @pl.when(jnp.logical_and(i >= 3, i - 3 < nc))
        def _retire():
            pltpu.make_async_copy(table.at[pl.ds(0, CH)], bufs[h3],
                                  gsems[h3]).wait()
            writeback(base + i - 3, bufs[h3], wsems[h3], wait=False)

        @pl.when(i < nc)
        def _issue():
            pltpu.async_copy(table.at[idx_v[i]], bufs[h], gsems[h])

    def loop_body(j, carry):
        for h in range(NBUF):
            step(NBUF * j, h) if h == 0 else step(NBUF * j + h, h)
        return carry

    lax.fori_loop(0, (NC_MAX + NBUF) // NBUF, loop_body, 0)

    @pl.when(wid < N_CTX)
    def _ctx_drain():
        for m in range(CTX_NWR):
            pltpu.make_async_copy(cbuf, out.at[pl.ds(0, CTX_BLK)], csem).wait()


def kernel(tokens, table, ctx):
    tokens = tokens.astype(jnp.int32)
    idx_flat = jnp.concatenate([
        tokens[:, 0],
        jnp.zeros((8,), jnp.int32),
        tokens[:, 1 + N_CTX:].T.reshape(-1),
        jnp.zeros((NW * NC_MAX * CH - (K + 8 + K * (N_TOK - N_CTX - 1)),),
                  jnp.int32),
    ])
    starts = np.arange(NW) * 119 + np.minimum(np.arange(NW), 5)
    gids = starts[:, None] * CH + np.arange(NC_MAX * CH)[None, :]
    idxw = idx_flat[jnp.asarray(gids)].reshape(NW, NC_MAX, CH)
    ctx_rep = jnp.broadcast_to(ctx[:, None, :], (N_CTX, CTX_BLK, DIM))

    mesh = plsc.VectorSubcoreMesh(core_axis_name="c", subcore_axis_name="s")
    run = pl.kernel(
        _sc_body,
        out_type=jax.ShapeDtypeStruct((N_TOK * K, DIM), jnp.float32),
        mesh=mesh,
        scratch_types=[
            pltpu.VMEM((NC_MAX, CH), jnp.int32),
            pltpu.VMEM((CTX_BLK, DIM), jnp.float32),
        ] + [pltpu.VMEM((CH, DIM), jnp.float32)] * NBUF
          + [pltpu.SemaphoreType.DMA] * (2 * NBUF + 1),
    )
    out = run(idxw, table, ctx_rep)
    return out.reshape(N_TOK, K, DIM).transpose(1, 0, 2)


# speedup vs baseline: 9.8848x; 1.0768x over previous
import functools

import jax
import jax.numpy as jnp
import numpy as np
from jax import lax
from jax.experimental import pallas as pl
from jax.experimental.pallas import tpu as pltpu
from jax.experimental.pallas import tpu_sc as plsc

VOCAB = 49408
K = 1000
N_TOK = 77
N_CTX = 16
DIM = 768

NW = 32
CH = 16
NA = 64
NB = 3750
NB_PAD = 3776
NCH = NA + NB
SCHUNK = 62
ECHUNK = 63
NC = 120
NBUF = 6
CTX_BLK = 40
CTX_NWR = K * N_CTX // (16 * CTX_BLK)


def _sc_body(idx_a, idx_b, table, ctxr, out, idx_va, idx_vb, cbuf, *scr):
    bufs = scr[:NBUF]
    gsems = scr[NBUF:2 * NBUF]
    wsems = scr[2 * NBUF:3 * NBUF]
    csem = scr[3 * NBUF]
    wid = lax.axis_index("s") * 2 + lax.axis_index("c")
    base = wid * NC
    nc = jnp.minimum(NC, NCH - base)

    @pl.when(wid == 0)
    def _stage0():
        pltpu.sync_copy(idx_a, idx_va)
        pltpu.sync_copy(idx_b.at[pl.ds(0, NC)], idx_vb)

    @pl.when(wid > 0)
    def _stage():
        off = pl.multiple_of(base - NA, 8)
        pltpu.sync_copy(idx_b.at[pl.ds(off, NC)], idx_vb)

    @pl.when(wid < N_CTX)
    def _ctx():
        pltpu.sync_copy(ctxr.at[wid], cbuf)
        for m in range(CTX_NWR):
            r = pl.multiple_of(K * (wid + 1) + CTX_BLK * m, 8)
            pltpu.async_copy(cbuf, out.at[pl.ds(r, CTX_BLK)], csem)

    def writeback(c, buf, wsem, wait):
        r16 = pl.multiple_of(
            jnp.where(c < SCHUNK, CH * c, CH * c + (N_CTX + 1) * K - NA * CH), 8)
        d16 = pltpu.make_async_copy(buf, out.at[pl.ds(r16, CH)], wsem)
        d8 = pltpu.make_async_copy(buf.at[pl.ds(0, 8)],
                                   out.at[pl.ds(SCHUNK * CH, 8)], wsem)

        @pl.when(c == SCHUNK)
        def _():
            d8.wait() if wait else d8.start()

        @pl.when(c != SCHUNK)
        def _():
            d16.wait() if wait else d16.start()

    def live(i):
        return jnp.logical_and(i < nc, base + i != ECHUNK)

    def step(i, h):
        @pl.when(jnp.logical_and(i >= 6, live(i - 6)))
        def _drain():
            writeback(base + i - 6, bufs[h], wsems[h], wait=True)

        h3 = (h + 3) % NBUF

        @pl.when(jnp.logical_and(i >= 3, live(i - 3)))
        def _retire():
            pltpu.make_async_copy(table.at[pl.ds(0, CH)], bufs[h3],
                                  gsems[h3]).wait()
            writeback(base + i - 3, bufs[h3], wsems[h3], wait=False)

        @pl.when(live(i))
        def _issue():
            c = base + i

            @pl.when(c < NA)
            def _a():
                pltpu.async_copy(table.at[idx_va[i]], bufs[h], gsems[h])

            @pl.when(c >= NA)
            def _b():
                pltpu.async_copy(table.at[idx_vb[jnp.where(wid == 0, i - NA, i)]],
                                 bufs[h], gsems[h])

    def loop_body(j, carry):
        for h in range(NBUF):
            step(NBUF * j + h, h)
        return carry

    lax.fori_loop(0, (NC + NBUF) // NBUF, loop_body, 0)

    @pl.when(wid < N_CTX)
    def _ctx_drain():
        for m in range(CTX_NWR):
            pltpu.make_async_copy(cbuf, out.at[pl.ds(0, CTX_BLK)], csem).wait()


def kernel(tokens, table, ctx):
    tokens = tokens.astype(jnp.int32)
    idx_a = jnp.zeros((NA * CH,), jnp.int32).at[:K].set(
        tokens[:, 0]).reshape(NA, CH)
    idx_b = jnp.zeros((NB_PAD, CH), jnp.int32).at[:NB].set(
        tokens[:, 1 + N_CTX:].T.reshape(NB, CH))
    ctx_rep = jnp.broadcast_to(ctx[:, None, :], (N_CTX, CTX_BLK, DIM))

    mesh = plsc.VectorSubcoreMesh(core_axis_name="c", subcore_axis_name="s")
    run = pl.kernel(
        _sc_body,
        out_type=jax.ShapeDtypeStruct((N_TOK * K, DIM), jnp.float32),
        mesh=mesh,
        scratch_types=[
            pltpu.VMEM((NA, CH), jnp.int32),
            pltpu.VMEM((NC, CH), jnp.int32),
            pltpu.VMEM((CTX_BLK, DIM), jnp.float32),
        ] + [pltpu.VMEM((CH, DIM), jnp.float32)] * NBUF
          + [pltpu.SemaphoreType.DMA] * (2 * NBUF + 1),
    )
    out = run(idx_a, idx_b, table, ctx_rep)
    return out.reshape(N_TOK, K, DIM).transpose(1, 0, 2)


# speedup vs baseline: 10.4644x; 1.0586x over previous
import functools

import jax
import jax.numpy as jnp
import numpy as np
from jax import lax
from jax.experimental import pallas as pl
from jax.experimental.pallas import tpu as pltpu
from jax.experimental.pallas import tpu_sc as plsc

VOCAB = 49408
K = 1000
N_TOK = 77
N_CTX = 16
DIM = 768

NW = 32
CH = 16
NA = 64
NB = 3750
NB_PAD = 3776
NCH = NA + NB
SCHUNK = 62
ECHUNK = 63
NC_CTX = 88
NC_GAT = 152
NBUF = 5
CTX_BLK = 40
CTX_NWR = K * N_CTX // (16 * CTX_BLK)


def _sc_body(idx_a, idx_b, table, ctxr, out, idx_va, idx_vb, cbuf, *scr):
    bufs = scr[:NBUF]
    gsems = scr[NBUF:2 * NBUF]
    wsems = scr[2 * NBUF:3 * NBUF]
    csem = scr[3 * NBUF]
    wid = lax.axis_index("s") * 2 + lax.axis_index("c")
    base = jnp.where(wid < N_CTX, NC_CTX * wid,
                     N_CTX * NC_CTX + NC_GAT * (wid - N_CTX))
    cap = jnp.where(wid < N_CTX, NC_CTX, NC_GAT)
    nc = jnp.clip(NCH - base, 0, cap)

    @pl.when(wid == 0)
    def _stage0():
        pltpu.sync_copy(idx_a, idx_va)
        pltpu.sync_copy(idx_b.at[pl.ds(0, NC_GAT)], idx_vb)

    @pl.when(wid > 0)
    def _stage():
        off = pl.multiple_of(base - NA, 8)
        pltpu.sync_copy(idx_b.at[pl.ds(off, NC_GAT)], idx_vb)

    @pl.when(wid < N_CTX)
    def _ctx():
        pltpu.sync_copy(ctxr.at[wid], cbuf)
        for m in range(CTX_NWR):
            r = pl.multiple_of(K * (wid + 1) + CTX_BLK * m, 8)
            pltpu.async_copy(cbuf, out.at[pl.ds(r, CTX_BLK)], csem)

    def writeback(c, buf, wsem, wait):
        r16 = pl.multiple_of(
            jnp.where(c < SCHUNK, CH * c, CH * c + (N_CTX + 1) * K - NA * CH), 8)
        d16 = pltpu.make_async_copy(buf, out.at[pl.ds(r16, CH)], wsem)
        d8 = pltpu.make_async_copy(buf.at[pl.ds(0, 8)],
                                   out.at[pl.ds(SCHUNK * CH, 8)], wsem)

        @pl.when(c == SCHUNK)
        def _():
            d8.wait() if wait else d8.start()

        @pl.when(c != SCHUNK)
        def _():
            d16.wait() if wait else d16.start()

    def live(i):
        return jnp.logical_and(i < nc, base + i != ECHUNK)

    def step(i, h):
        @pl.when(jnp.logical_and(i >= NBUF, live(i - NBUF)))
        def _drain():
            writeback(base + i - NBUF, bufs[h], wsems[h], wait=True)

        h3 = (h + NBUF - 3) % NBUF

        @pl.when(jnp.logical_and(i >= 3, live(i - 3)))
        def _retire():
            pltpu.make_async_copy(table.at[pl.ds(0, CH)], bufs[h3],
                                  gsems[h3]).wait()
            writeback(base + i - 3, bufs[h3], wsems[h3], wait=False)

        @pl.when(live(i))
        def _issue():
            c = base + i

            @pl.when(c < NA)
            def _a():
                pltpu.async_copy(table.at[idx_va[i]], bufs[h], gsems[h])

            @pl.when(c >= NA)
            def _b():
                pltpu.async_copy(table.at[idx_vb[jnp.where(wid == 0, i - NA, i)]],
                                 bufs[h], gsems[h])

    def loop_body(j, carry):
        for h in range(NBUF):
            step(NBUF * j + h, h)
        return carry

    lax.fori_loop(0, (NC_GAT + 2 * NBUF) // NBUF, loop_body, 0)

    @pl.when(wid < N_CTX)
    def _ctx_drain():
        for m in range(CTX_NWR):
            pltpu.make_async_copy(cbuf, out.at[pl.ds(0, CTX_BLK)], csem).wait()


def kernel(tokens, table, ctx):
    tokens = tokens.astype(jnp.int32)
    idx_a = jnp.zeros((NA * CH,), jnp.int32).at[:K].set(
        tokens[:, 0]).reshape(NA, CH)
    idx_b = jnp.zeros((NB_PAD, CH), jnp.int32).at[:NB].set(
        tokens[:, 1 + N_CTX:].T.reshape(NB, CH))
    ctx_rep = jnp.broadcast_to(ctx[:, None, :], (N_CTX, CTX_BLK, DIM))

    mesh = plsc.VectorSubcoreMesh(core_axis_name="c", subcore_axis_name="s")
    run = pl.kernel(
        _sc_body,
        out_type=jax.ShapeDtypeStruct((N_TOK * K, DIM), jnp.float32),
        mesh=mesh,
        scratch_types=[
            pltpu.VMEM((NA, CH), jnp.int32),
            pltpu.VMEM((NC_GAT, CH), jnp.int32),
            pltpu.VMEM((CTX_BLK, DIM), jnp.float32),
        ] + [pltpu.VMEM((CH, DIM), jnp.float32)] * NBUF
          + [pltpu.SemaphoreType.DMA] * (2 * NBUF + 1),
    )
    out = run(idx_a, idx_b, table, ctx_rep)
    return out.reshape(N_TOK, K, DIM).transpose(1, 0, 2)
